# baseline (speedup 1.0000x reference)
import functools

import jax
import jax.numpy as jnp
from jax import lax
from jax.experimental import pallas as pl
from jax.experimental.pallas import tpu as pltpu
from jax.experimental.pallas import tpu_sc as plsc

_N = 10000
_D = 128
_EPS = 1e-5

_NC = 2
_NS = 16
_NW = _NC * _NS
_CH = 80
_NP = 10240
_RPT = _NP // _NS


def _ln(x, g, b):
    m = jnp.mean(x, axis=-1, keepdims=True)
    v = jnp.mean((x - m) ** 2, axis=-1, keepdims=True)
    return (x - m) / jnp.sqrt(v + _EPS) * g + b


def _mlp_body(tea, p1, pb1, g, b, p2r, pb2, out):
    h = jnp.dot(tea[...], p1[...], preferred_element_type=jnp.float32) + pb1[...]
    h = jnp.maximum(_ln(h, g[...], b[...]), 0.0)
    out[...] = jnp.sum(h * p2r[...], axis=-1, keepdims=True) + pb2[...]


def _edge_mlp(tea, p1, pb1, g, b, p2, pb2):
    e, k = tea.shape
    tea_p = jnp.pad(tea, ((0, 0), (0, _D - k)))
    p1_p = jnp.pad(p1, ((0, _D - k), (0, 0)))
    be = 4000
    return pl.pallas_call(
        _mlp_body,
        grid=(e // be,),
        in_specs=[
            pl.BlockSpec((be, _D), lambda i: (i, 0)),
            pl.BlockSpec((_D, _D), lambda i: (0, 0)),
            pl.BlockSpec((1, _D), lambda i: (0, 0)),
            pl.BlockSpec((1, _D), lambda i: (0, 0)),
            pl.BlockSpec((1, _D), lambda i: (0, 0)),
            pl.BlockSpec((1, _D), lambda i: (0, 0)),
            pl.BlockSpec((1, 1), lambda i: (0, 0)),
        ],
        out_specs=pl.BlockSpec((be, 1), lambda i: (i, 0)),
        out_shape=jax.ShapeDtypeStruct((e, 1), jnp.float32),
    )(tea_p, p1_p, pb1.reshape(1, _D), g.reshape(1, _D), b.reshape(1, _D),
      p2.reshape(1, _D), pb2.reshape(1, 1))


def _sc_degree(d2, w2):
    nchunk = d2.shape[1]

    @functools.partial(
        pl.kernel,
        out_type=jax.ShapeDtypeStruct((_NC, _NP), jnp.float32),
        mesh=plsc.VectorSubcoreMesh(core_axis_name="c", subcore_axis_name="s"),
        scratch_types=[
            pltpu.VMEM((nchunk, _CH), jnp.int32),
            pltpu.VMEM((nchunk, _CH), jnp.float32),
            pltpu.VMEM((_RPT,), jnp.float32),
            pltpu.VMEM_SHARED((_NP,), jnp.float32),
        ],
    )
    def k(d_h, w_h, out_h, dix, wv, zb, acc):
        c = lax.axis_index("c")
        s = lax.axis_index("s")
        wid = s * _NC + c
        for i in range(_RPT // 16):
            zb[pl.ds(i * 16, 16)] = jnp.zeros((16,), jnp.float32)
        pltpu.sync_copy(zb, acc.at[pl.ds(s * _RPT, _RPT)])
        pltpu.sync_copy(d_h.at[wid], dix)
        pltpu.sync_copy(w_h.at[wid], wv)
        plsc.subcore_barrier()

        def chunk(g, carry):
            pltpu.sync_copy(wv.at[g], acc.at[dix.at[g]], add=True)
            return carry

        lax.fori_loop(0, nchunk, chunk, 0)
        plsc.subcore_barrier()
        pltpu.sync_copy(acc.at[pl.ds(s * _RPT, _RPT)],
                        out_h.at[c, pl.ds(s * _RPT, _RPT)])

    return k(d2, w2)


def _sc_messages(hp, rec, w2):
    nchunk = rec.shape[1]

    @functools.partial(
        pl.kernel,
        out_type=jax.ShapeDtypeStruct((_NC, _NP, _D), jnp.float32),
        mesh=plsc.VectorSubcoreMesh(core_axis_name="c", subcore_axis_name="s"),
        scratch_types=[
            pltpu.VMEM((4, 2, _CH), jnp.int32),
            pltpu.VMEM((4, _CH), jnp.float32),
            pltpu.VMEM((2, _CH, _D), jnp.float32),
            pltpu.VMEM_SHARED((_NP, _D), jnp.float32),
            [pltpu.SemaphoreType.DMA] * 2,
            [pltpu.SemaphoreType.DMA] * 2,
            [pltpu.SemaphoreType.DMA] * 4,
        ],
    )
    def k(hp_h, rec_h, w_h, out_h, e3, wv, rows, acc, gsem, ssem, rsem):
        c = lax.axis_index("c")
        s = lax.axis_index("s")
        wid = s * _NC + c

        def zrow(e, carry):
            for j in range(_D // 16):
                rows[0, e, pl.ds(j * 16, 16)] = jnp.zeros((16,), jnp.float32)
            return carry

        lax.fori_loop(0, _CH, zrow, 0)
        for kk in range(_RPT // _CH):
            pltpu.sync_copy(rows.at[0],
                            acc.at[pl.ds(s * _RPT + kk * _CH, _CH)])
        plsc.subcore_barrier()

        def refill_start(g, ep):
            pltpu.async_copy(rec_h.at[wid, g], e3.at[ep], rsem[ep])
            pltpu.async_copy(w_h.at[wid, g], wv.at[ep], rsem[ep])

        def refill_wait(g, ep):
            pltpu.make_async_copy(rec_h.at[wid, g], e3.at[ep],
                                  rsem[ep]).wait()
            pltpu.make_async_copy(w_h.at[wid, g], wv.at[ep],
                                  rsem[ep]).wait()

        def gather_start(rp, ep):
            pltpu.async_copy(hp_h.at[e3.at[ep, 0]], rows.at[rp], gsem[rp])

        def gather_wait(rp, ep):
            pltpu.make_async_copy(hp_h.at[e3.at[ep, 0]], rows.at[rp],
                                  gsem[rp]).wait()

        def scatter_start(rp, ep):
            pltpu.async_copy(rows.at[rp], acc.at[e3.at[ep, 1]], ssem[rp],
                             add=True)

        def scatter_wait(rp, ep):
            pltpu.make_async_copy(rows.at[rp], acc.at[e3.at[ep, 1]],
                                  ssem[rp]).wait()

        def scale(rp, ep):
            def body(kk, cc):
                w16 = wv[ep, pl.ds(kk * 16, 16)]
                base = kk * 16
                for l in range(16):
                    wvec = jnp.full((16,), w16[l])
                    for j in range(_D // 16):
                        sl = pl.ds(j * 16, 16)
                        rows[rp, base + l, sl] = rows[rp, base + l, sl] * wvec
                return cc

            lax.fori_loop(0, _CH // 16, body, 0)

        for g in range(3):
            refill_start(g, g)
        refill_wait(0, 0)
        gather_start(0, 0)

        def quad(i, carry):
            for kph in range(4):
                ch = 4 * i + kph
                rp = kph % 2
                ep = kph
                epn = (kph + 1) % 4
                epr = (kph + 3) % 4
                gather_wait(rp, ep)

                @pl.when(ch > 0)
                def _():
                    scatter_wait(1 - rp, (kph + 3) % 4)

                @pl.when(ch + 1 < nchunk)
                def _():
                    refill_wait(ch + 1, epn)
                    gather_start(1 - rp, epn)

                scale(rp, ep)
                scatter_start(rp, ep)

                @pl.when(ch + 3 < nchunk)
                def _():
                    refill_start(ch + 3, epr)

            return carry

        lax.fori_loop(0, nchunk // 4, quad, 0)
        if nchunk % 4 == 1:
            gather_wait(0, 0)
            scatter_wait(1, 3)
            scale(0, 0)
            pltpu.sync_copy(rows.at[0], acc.at[e3.at[0, 1]], add=True)
        plsc.subcore_barrier()
        pltpu.sync_copy(acc.at[pl.ds(s * _RPT, _RPT)],
                        out_h.at[c, pl.ds(s * _RPT, _RPT)])

    return k(hp, rec, w2)


def _prep_body(degp, x, wb, h1p, d1o):
    db = degp[0] + degp[1] + 1.0
    d1 = jnp.where(db > 0, 1.0 / jnp.sqrt(db), 0.0)
    d1o[...] = d1
    h1p[...] = d1 * jnp.dot(x[...], wb[...], preferred_element_type=jnp.float32)


def _prep(degp, x, wb):
    r = 400
    return pl.pallas_call(
        _prep_body,
        grid=(_N // r,),
        in_specs=[
            pl.BlockSpec((2, r, 1), lambda i: (0, i, 0)),
            pl.BlockSpec((r, _D), lambda i: (i, 0)),
            pl.BlockSpec((_D, _D), lambda i: (0, 0)),
        ],
        out_specs=[
            pl.BlockSpec((r, _D), lambda i: (i, 0)),
            pl.BlockSpec((r, 1), lambda i: (i, 0)),
        ],
        out_shape=[
            jax.ShapeDtypeStruct((_N, _D), jnp.float32),
            jax.ShapeDtypeStruct((_N, 1), jnp.float32),
        ],
    )(degp, x, wb)


def _mid_body(sp, h1p, d1, degt, x, bb, sg, sb, wt, s2o, h2po, d2o):
    u = d1[...] * (sp[0] + sp[1] + h1p[...]) + bb[...] + x[...]
    u = jnp.maximum(_ln(u, sg[...], sb[...]), 0.0)
    s2o[...] = u
    dt = degt[0] + degt[1] + 1.0
    d2 = jnp.where(dt > 0, 1.0 / jnp.sqrt(dt), 0.0)
    d2o[...] = d2
    h2po[...] = d2 * jnp.dot(u, wt[...], preferred_element_type=jnp.float32)


def _mid(sp, h1p, d1, degt, x, bb, sg, sb, wt):
    r = 400
    return pl.pallas_call(
        _mid_body,
        grid=(_N // r,),
        in_specs=[
            pl.BlockSpec((2, r, _D), lambda i: (0, i, 0)),
            pl.BlockSpec((r, _D), lambda i: (i, 0)),
            pl.BlockSpec((r, 1), lambda i: (i, 0)),
            pl.BlockSpec((2, r, 1), lambda i: (0, i, 0)),
            pl.BlockSpec((r, _D), lambda i: (i, 0)),
            pl.BlockSpec((1, _D), lambda i: (0, 0)),
            pl.BlockSpec((1, _D), lambda i: (0, 0)),
            pl.BlockSpec((1, _D), lambda i: (0, 0)),
            pl.BlockSpec((_D, _D), lambda i: (0, 0)),
        ],
        out_specs=[
            pl.BlockSpec((r, _D), lambda i: (i, 0)),
            pl.BlockSpec((r, _D), lambda i: (i, 0)),
            pl.BlockSpec((r, 1), lambda i: (i, 0)),
        ],
        out_shape=[
            jax.ShapeDtypeStruct((_N, _D), jnp.float32),
            jax.ShapeDtypeStruct((_N, _D), jnp.float32),
            jax.ShapeDtypeStruct((_N, 1), jnp.float32),
        ],
    )(sp, h1p, d1, degt, x, bb.reshape(1, _D), sg.reshape(1, _D),
      sb.reshape(1, _D), wt)


def _final_body(sp, h2p, d2, s2, x, bt, tg, tb, sg, sb, out):
    u = d2[...] * (sp[0] + sp[1] + h2p[...]) + bt[...] + s2[...]
    u = jnp.maximum(_ln(u, tg[...], tb[...]), 0.0)
    out[...] = _ln(u + x[...], sg[...], sb[...])


def _final(sp, h2p, d2, s2, x, bt, tg, tb, sg, sb):
    r = 400
    return pl.pallas_call(
        _final_body,
        grid=(_N // r,),
        in_specs=[
            pl.BlockSpec((2, r, _D), lambda i: (0, i, 0)),
            pl.BlockSpec((r, _D), lambda i: (i, 0)),
            pl.BlockSpec((r, 1), lambda i: (i, 0)),
            pl.BlockSpec((r, _D), lambda i: (i, 0)),
            pl.BlockSpec((r, _D), lambda i: (i, 0)),
            pl.BlockSpec((1, _D), lambda i: (0, 0)),
            pl.BlockSpec((1, _D), lambda i: (0, 0)),
            pl.BlockSpec((1, _D), lambda i: (0, 0)),
            pl.BlockSpec((1, _D), lambda i: (0, 0)),
            pl.BlockSpec((1, _D), lambda i: (0, 0)),
        ],
        out_specs=pl.BlockSpec((r, _D), lambda i: (i, 0)),
        out_shape=jax.ShapeDtypeStruct((_N, _D), jnp.float32),
    )(sp, h2p, d2, s2, x, bt.reshape(1, _D), tg.reshape(1, _D),
      tb.reshape(1, _D), sg.reshape(1, _D), sb.reshape(1, _D))


def kernel(x_struct, x_gene, bold_edge_index, bold_edge_attr,
           temporal_edge_index, temporal_edge_attr, time_steps,
           W_bold, b_bold, W_temp, b_temp, P1, pb1, ln_g, ln_b, P2, pb2,
           struct_g, struct_b, temp_g, temp_b):
    bw = bold_edge_attr[:, 0]

    bs2 = bold_edge_index[0].reshape(_NW, -1, _CH)
    bd2 = bold_edge_index[1].reshape(_NW, -1, _CH)
    ts2 = temporal_edge_index[0].reshape(_NW, -1, _CH)
    td2 = temporal_edge_index[1].reshape(_NW, -1, _CH)
    bw2 = bw.reshape(_NW, -1, _CH)

    brec = jnp.stack([bs2, bd2], axis=2)

    degb = _sc_degree(bd2, bw2)[:, :, None]
    h1p, d1 = _prep(degb, x_struct, W_bold)
    sb_p = _sc_messages(h1p, brec, bw2)

    tw = _edge_mlp(temporal_edge_attr, P1, pb1, ln_g, ln_b, P2, pb2)[:, 0]
    tw2 = tw.reshape(_NW, -1, _CH)
    trec = jnp.stack([ts2, td2], axis=2)
    tw2b, _ = lax.optimization_barrier((tw2, sb_p))
    degt = _sc_degree(td2, tw2b)[:, :, None]

    struct2, h2p, d2 = _mid(sb_p, h1p, d1, degt, x_struct, b_bold,
                            struct_g, struct_b, W_temp)

    st_p = _sc_messages(h2p, trec, tw2)
    out = _final(st_p, h2p, d2, struct2, x_struct, b_temp,
                 temp_g, temp_b, struct_g, struct_b)
    return (out, out)


# speedup vs baseline: 1.0367x; 1.0367x over previous
import functools

import jax
import jax.numpy as jnp
from jax import lax
from jax.experimental import pallas as pl
from jax.experimental.pallas import tpu as pltpu
from jax.experimental.pallas import tpu_sc as plsc

_N = 10000
_D = 128
_EPS = 1e-5

_NC = 2
_NS = 16
_NW = _NC * _NS
_CH = 80
_NP = 10240
_RPT = _NP // _NS


def _ln(x, g, b):
    m = jnp.mean(x, axis=-1, keepdims=True)
    xm = x - m
    v = jnp.mean(xm * xm, axis=-1, keepdims=True)
    return xm * (1.0 / jnp.sqrt(v + _EPS)) * g + b


def _mlp_body(tea, p1, pb1, g, b, p2, pb2, out):
    h = jnp.dot(tea[...], p1[...], preferred_element_type=jnp.float32) + pb1[...]
    h = jnp.maximum(_ln(h, g[...], b[...]), 0.0)
    out[...] = jnp.dot(h, p2[...], preferred_element_type=jnp.float32) + pb2[...]


def _edge_mlp(tea, p1, pb1, g, b, p2, pb2):
    e, k = tea.shape
    be = 2000
    return pl.pallas_call(
        _mlp_body,
        grid=(e // be,),
        in_specs=[
            pl.BlockSpec((be, k), lambda i: (i, 0)),
            pl.BlockSpec((k, _D), lambda i: (0, 0)),
            pl.BlockSpec((1, _D), lambda i: (0, 0)),
            pl.BlockSpec((1, _D), lambda i: (0, 0)),
            pl.BlockSpec((1, _D), lambda i: (0, 0)),
            pl.BlockSpec((_D, 1), lambda i: (0, 0)),
            pl.BlockSpec((1, 1), lambda i: (0, 0)),
        ],
        out_specs=pl.BlockSpec((be, 1), lambda i: (i, 0)),
        out_shape=jax.ShapeDtypeStruct((e, 1), jnp.float32),
    )(tea, p1, pb1.reshape(1, _D), g.reshape(1, _D), b.reshape(1, _D),
      p2, pb2.reshape(1, 1))


def _sc_degree(d2, w2):
    nchunk = d2.shape[1]

    @functools.partial(
        pl.kernel,
        out_type=jax.ShapeDtypeStruct((_NC, _NP), jnp.float32),
        mesh=plsc.VectorSubcoreMesh(core_axis_name="c", subcore_axis_name="s"),
        scratch_types=[
            pltpu.VMEM((nchunk, _CH), jnp.int32),
            pltpu.VMEM((nchunk, _CH), jnp.float32),
            pltpu.VMEM((_RPT,), jnp.float32),
            pltpu.VMEM_SHARED((_NP,), jnp.float32),
        ],
    )
    def k(d_h, w_h, out_h, dix, wv, zb, acc):
        c = lax.axis_index("c")
        s = lax.axis_index("s")
        wid = s * _NC + c
        for i in range(_RPT // 16):
            zb[pl.ds(i * 16, 16)] = jnp.zeros((16,), jnp.float32)
        pltpu.sync_copy(zb, acc.at[pl.ds(s * _RPT, _RPT)])
        pltpu.sync_copy(d_h.at[wid], dix)
        pltpu.sync_copy(w_h.at[wid], wv)
        plsc.subcore_barrier()

        def chunk(g, carry):
            pltpu.sync_copy(wv.at[g], acc.at[dix.at[g]], add=True)
            return carry

        lax.fori_loop(0, nchunk, chunk, 0)
        plsc.subcore_barrier()
        pltpu.sync_copy(acc.at[pl.ds(s * _RPT, _RPT)],
                        out_h.at[c, pl.ds(s * _RPT, _RPT)])

    return k(d2, w2)


def _sc_messages(hp, rec, w2):
    nchunk = rec.shape[1]

    @functools.partial(
        pl.kernel,
        out_type=jax.ShapeDtypeStruct((_NC, _NP, _D), jnp.float32),
        mesh=plsc.VectorSubcoreMesh(core_axis_name="c", subcore_axis_name="s"),
        scratch_types=[
            pltpu.VMEM((4, 2, _CH), jnp.int32),
            pltpu.VMEM((4, _CH), jnp.float32),
            pltpu.VMEM((2, _CH, _D), jnp.float32),
            pltpu.VMEM_SHARED((_NP, _D), jnp.float32),
            [pltpu.SemaphoreType.DMA] * 2,
            [pltpu.SemaphoreType.DMA] * 2,
            [pltpu.SemaphoreType.DMA] * 4,
        ],
    )
    def k(hp_h, rec_h, w_h, out_h, e3, wv, rows, acc, gsem, ssem, rsem):
        c = lax.axis_index("c")
        s = lax.axis_index("s")
        wid = s * _NC + c

        def zrow(e, carry):
            for j in range(_D // 16):
                rows[0, e, pl.ds(j * 16, 16)] = jnp.zeros((16,), jnp.float32)
            return carry

        lax.fori_loop(0, _CH, zrow, 0)
        for kk in range(_RPT // _CH):
            pltpu.sync_copy(rows.at[0],
                            acc.at[pl.ds(s * _RPT + kk * _CH, _CH)])
        plsc.subcore_barrier()

        def refill_start(g, ep):
            pltpu.async_copy(rec_h.at[wid, g], e3.at[ep], rsem[ep])
            pltpu.async_copy(w_h.at[wid, g], wv.at[ep], rsem[ep])

        def refill_wait(g, ep):
            pltpu.make_async_copy(rec_h.at[wid, g], e3.at[ep],
                                  rsem[ep]).wait()
            pltpu.make_async_copy(w_h.at[wid, g], wv.at[ep],
                                  rsem[ep]).wait()

        def gather_start(rp, ep):
            pltpu.async_copy(hp_h.at[e3.at[ep, 0]], rows.at[rp], gsem[rp])

        def gather_wait(rp, ep):
            pltpu.make_async_copy(hp_h.at[e3.at[ep, 0]], rows.at[rp],
                                  gsem[rp]).wait()

        def scatter_start(rp, ep):
            pltpu.async_copy(rows.at[rp], acc.at[e3.at[ep, 1]], ssem[rp],
                             add=True)

        def scatter_wait(rp, ep):
            pltpu.make_async_copy(rows.at[rp], acc.at[e3.at[ep, 1]],
                                  ssem[rp]).wait()

        def scale(rp, ep):
            def body(kk, cc):
                w16 = wv[ep, pl.ds(kk * 16, 16)]
                base = kk * 16
                for l in range(16):
                    wvec = jnp.full((16,), w16[l])
                    for j in range(_D // 16):
                        sl = pl.ds(j * 16, 16)
                        rows[rp, base + l, sl] = rows[rp, base + l, sl] * wvec
                return cc

            lax.fori_loop(0, _CH // 16, body, 0)

        for g in range(3):
            refill_start(g, g)
        refill_wait(0, 0)
        gather_start(0, 0)

        def quad(i, carry):
            for kph in range(4):
                ch = 4 * i + kph
                rp = kph % 2
                ep = kph
                epn = (kph + 1) % 4
                epr = (kph + 3) % 4
                gather_wait(rp, ep)

                @pl.when(ch > 0)
                def _():
                    scatter_wait(1 - rp, (kph + 3) % 4)

                @pl.when(ch + 1 < nchunk)
                def _():
                    refill_wait(ch + 1, epn)
                    gather_start(1 - rp, epn)

                scale(rp, ep)
                scatter_start(rp, ep)

                @pl.when(ch + 3 < nchunk)
                def _():
                    refill_start(ch + 3, epr)

            return carry

        lax.fori_loop(0, nchunk // 4, quad, 0)
        if nchunk % 4 == 1:
            gather_wait(0, 0)
            scatter_wait(1, 3)
            scale(0, 0)
            pltpu.sync_copy(rows.at[0], acc.at[e3.at[0, 1]], add=True)
        plsc.subcore_barrier()
        pltpu.sync_copy(acc.at[pl.ds(s * _RPT, _RPT)],
                        out_h.at[c, pl.ds(s * _RPT, _RPT)])

    return k(hp, rec, w2)


def _prep_body(degp, x, wb, h1p, d1o):
    db = degp[0] + degp[1] + 1.0
    d1 = jnp.where(db > 0, 1.0 / jnp.sqrt(db), 0.0)
    d1o[...] = d1
    h1p[...] = d1 * jnp.dot(x[...], wb[...], preferred_element_type=jnp.float32)


def _prep(degp, x, wb):
    r = 400
    return pl.pallas_call(
        _prep_body,
        grid=(_N // r,),
        in_specs=[
            pl.BlockSpec((2, r, 1), lambda i: (0, i, 0)),
            pl.BlockSpec((r, _D), lambda i: (i, 0)),
            pl.BlockSpec((_D, _D), lambda i: (0, 0)),
        ],
        out_specs=[
            pl.BlockSpec((r, _D), lambda i: (i, 0)),
            pl.BlockSpec((r, 1), lambda i: (i, 0)),
        ],
        out_shape=[
            jax.ShapeDtypeStruct((_N, _D), jnp.float32),
            jax.ShapeDtypeStruct((_N, 1), jnp.float32),
        ],
    )(degp, x, wb)


def _mid_body(sp, h1p, d1, degt, x, bb, sg, sb, wt, s2o, h2po, d2o):
    u = d1[...] * (sp[0] + sp[1] + h1p[...]) + bb[...] + x[...]
    u = jnp.maximum(_ln(u, sg[...], sb[...]), 0.0)
    s2o[...] = u
    dt = degt[0] + degt[1] + 1.0
    d2 = jnp.where(dt > 0, 1.0 / jnp.sqrt(dt), 0.0)
    d2o[...] = d2
    h2po[...] = d2 * jnp.dot(u, wt[...], preferred_element_type=jnp.float32)


def _mid(sp, h1p, d1, degt, x, bb, sg, sb, wt):
    r = 400
    return pl.pallas_call(
        _mid_body,
        grid=(_N // r,),
        in_specs=[
            pl.BlockSpec((2, r, _D), lambda i: (0, i, 0)),
            pl.BlockSpec((r, _D), lambda i: (i, 0)),
            pl.BlockSpec((r, 1), lambda i: (i, 0)),
            pl.BlockSpec((2, r, 1), lambda i: (0, i, 0)),
            pl.BlockSpec((r, _D), lambda i: (i, 0)),
            pl.BlockSpec((1, _D), lambda i: (0, 0)),
            pl.BlockSpec((1, _D), lambda i: (0, 0)),
            pl.BlockSpec((1, _D), lambda i: (0, 0)),
            pl.BlockSpec((_D, _D), lambda i: (0, 0)),
        ],
        out_specs=[
            pl.BlockSpec((r, _D), lambda i: (i, 0)),
            pl.BlockSpec((r, _D), lambda i: (i, 0)),
            pl.BlockSpec((r, 1), lambda i: (i, 0)),
        ],
        out_shape=[
            jax.ShapeDtypeStruct((_N, _D), jnp.float32),
            jax.ShapeDtypeStruct((_N, _D), jnp.float32),
            jax.ShapeDtypeStruct((_N, 1), jnp.float32),
        ],
    )(sp, h1p, d1, degt, x, bb.reshape(1, _D), sg.reshape(1, _D),
      sb.reshape(1, _D), wt)


def _final_body(sp, h2p, d2, s2, x, bt, tg, tb, sg, sb, out):
    u = d2[...] * (sp[0] + sp[1] + h2p[...]) + bt[...] + s2[...]
    u = jnp.maximum(_ln(u, tg[...], tb[...]), 0.0)
    out[...] = _ln(u + x[...], sg[...], sb[...])


def _final(sp, h2p, d2, s2, x, bt, tg, tb, sg, sb):
    r = 400
    return pl.pallas_call(
        _final_body,
        grid=(_N // r,),
        in_specs=[
            pl.BlockSpec((2, r, _D), lambda i: (0, i, 0)),
            pl.BlockSpec((r, _D), lambda i: (i, 0)),
            pl.BlockSpec((r, 1), lambda i: (i, 0)),
            pl.BlockSpec((r, _D), lambda i: (i, 0)),
            pl.BlockSpec((r, _D), lambda i: (i, 0)),
            pl.BlockSpec((1, _D), lambda i: (0, 0)),
            pl.BlockSpec((1, _D), lambda i: (0, 0)),
            pl.BlockSpec((1, _D), lambda i: (0, 0)),
            pl.BlockSpec((1, _D), lambda i: (0, 0)),
            pl.BlockSpec((1, _D), lambda i: (0, 0)),
        ],
        out_specs=pl.BlockSpec((r, _D), lambda i: (i, 0)),
        out_shape=jax.ShapeDtypeStruct((_N, _D), jnp.float32),
    )(sp, h2p, d2, s2, x, bt.reshape(1, _D), tg.reshape(1, _D),
      tb.reshape(1, _D), sg.reshape(1, _D), sb.reshape(1, _D))


def kernel(x_struct, x_gene, bold_edge_index, bold_edge_attr,
           temporal_edge_index, temporal_edge_attr, time_steps,
           W_bold, b_bold, W_temp, b_temp, P1, pb1, ln_g, ln_b, P2, pb2,
           struct_g, struct_b, temp_g, temp_b):
    bw = bold_edge_attr[:, 0]

    bs2 = bold_edge_index[0].reshape(_NW, -1, _CH)
    bd2 = bold_edge_index[1].reshape(_NW, -1, _CH)
    ts2 = temporal_edge_index[0].reshape(_NW, -1, _CH)
    td2 = temporal_edge_index[1].reshape(_NW, -1, _CH)
    bw2 = bw.reshape(_NW, -1, _CH)

    brec = jnp.stack([bs2, bd2], axis=2)

    degb = _sc_degree(bd2, bw2)[:, :, None]
    h1p, d1 = _prep(degb, x_struct, W_bold)
    sb_p = _sc_messages(h1p, brec, bw2)

    tw = _edge_mlp(temporal_edge_attr, P1, pb1, ln_g, ln_b, P2, pb2)[:, 0]
    tw2 = tw.reshape(_NW, -1, _CH)
    trec = jnp.stack([ts2, td2], axis=2)
    tw2b, _ = lax.optimization_barrier((tw2, sb_p))
    degt = _sc_degree(td2, tw2b)[:, :, None]

    struct2, h2p, d2 = _mid(sb_p, h1p, d1, degt, x_struct, b_bold,
                            struct_g, struct_b, W_temp)

    st_p = _sc_messages(h2p, trec, tw2)
    out = _final(st_p, h2p, d2, struct2, x_struct, b_temp,
                 temp_g, temp_b, struct_g, struct_b)
    return (out, out)


# speedup vs baseline: 1.1381x; 1.0979x over previous
import functools

import jax
import jax.numpy as jnp
from jax import lax
from jax.experimental import pallas as pl
from jax.experimental.pallas import tpu as pltpu
from jax.experimental.pallas import tpu_sc as plsc

_N = 10000
_D = 128
_EPS = 1e-5

_NC = 2
_NS = 16
_NW = _NC * _NS
_CH = 80
_NP = 10240
_RPT = _NP // _NS


def _ln(x, g, b):
    m = jnp.mean(x, axis=-1, keepdims=True)
    xm = x - m
    v = jnp.mean(xm * xm, axis=-1, keepdims=True)
    return xm * (1.0 / jnp.sqrt(v + _EPS)) * g + b


def _mlp_body(tea, p1, pb1, g, b, p2, pb2, out):
    h = jnp.dot(tea[...], p1[...], preferred_element_type=jnp.float32) + pb1[...]
    h = jnp.maximum(_ln(h, g[...], b[...]), 0.0)
    out[...] = jnp.dot(h, p2[...], preferred_element_type=jnp.float32) + pb2[...]


def _edge_mlp(tea, p1, pb1, g, b, p2, pb2):
    e, k = tea.shape
    be = 8000
    return pl.pallas_call(
        _mlp_body,
        grid=(e // be,),
        in_specs=[
            pl.BlockSpec((be, k), lambda i: (i, 0)),
            pl.BlockSpec((k, _D), lambda i: (0, 0)),
            pl.BlockSpec((1, _D), lambda i: (0, 0)),
            pl.BlockSpec((1, _D), lambda i: (0, 0)),
            pl.BlockSpec((1, _D), lambda i: (0, 0)),
            pl.BlockSpec((_D, 1), lambda i: (0, 0)),
            pl.BlockSpec((1, 1), lambda i: (0, 0)),
        ],
        out_specs=pl.BlockSpec((be, 1), lambda i: (i, 0)),
        out_shape=jax.ShapeDtypeStruct((e, 1), jnp.float32),
    )(tea, p1, pb1.reshape(1, _D), g.reshape(1, _D), b.reshape(1, _D),
      p2, pb2.reshape(1, 1))


def _sc_degree(d2, w2):
    nchunk = d2.shape[1]

    @functools.partial(
        pl.kernel,
        out_type=jax.ShapeDtypeStruct((_NC, _NP), jnp.float32),
        mesh=plsc.VectorSubcoreMesh(core_axis_name="c", subcore_axis_name="s"),
        scratch_types=[
            pltpu.VMEM((nchunk, _CH), jnp.int32),
            pltpu.VMEM((nchunk, _CH), jnp.float32),
            pltpu.VMEM((_RPT,), jnp.float32),
            pltpu.VMEM_SHARED((_NP,), jnp.float32),
        ],
    )
    def k(d_h, w_h, out_h, dix, wv, zb, acc):
        c = lax.axis_index("c")
        s = lax.axis_index("s")
        wid = s * _NC + c
        for i in range(_RPT // 16):
            zb[pl.ds(i * 16, 16)] = jnp.zeros((16,), jnp.float32)
        pltpu.sync_copy(zb, acc.at[pl.ds(s * _RPT, _RPT)])
        pltpu.sync_copy(d_h.at[wid], dix)
        pltpu.sync_copy(w_h.at[wid], wv)
        plsc.subcore_barrier()

        def chunk(g, carry):
            pltpu.sync_copy(wv.at[g], acc.at[dix.at[g]], add=True)
            return carry

        lax.fori_loop(0, nchunk, chunk, 0)
        plsc.subcore_barrier()
        pltpu.sync_copy(acc.at[pl.ds(s * _RPT, _RPT)],
                        out_h.at[c, pl.ds(s * _RPT, _RPT)])

    return k(d2, w2)


def _sc_messages(hp, rec, w2):
    nchunk = rec.shape[1]

    @functools.partial(
        pl.kernel,
        out_type=jax.ShapeDtypeStruct((_NC, _NP, _D), jnp.float32),
        mesh=plsc.VectorSubcoreMesh(core_axis_name="c", subcore_axis_name="s"),
        scratch_types=[
            pltpu.VMEM((4, 2, _CH), jnp.int32),
            pltpu.VMEM((4, _CH), jnp.float32),
            pltpu.VMEM((2, _CH, _D), jnp.float32),
            pltpu.VMEM_SHARED((_NP, _D), jnp.float32),
            [pltpu.SemaphoreType.DMA] * 2,
            [pltpu.SemaphoreType.DMA] * 2,
            [pltpu.SemaphoreType.DMA] * 4,
        ],
    )
    def k(hp_h, rec_h, w_h, out_h, e3, wv, rows, acc, gsem, ssem, rsem):
        c = lax.axis_index("c")
        s = lax.axis_index("s")
        wid = s * _NC + c

        def zrow(e, carry):
            for j in range(_D // 16):
                rows[0, e, pl.ds(j * 16, 16)] = jnp.zeros((16,), jnp.float32)
            return carry

        lax.fori_loop(0, _CH, zrow, 0)
        for kk in range(_RPT // _CH):
            pltpu.sync_copy(rows.at[0],
                            acc.at[pl.ds(s * _RPT + kk * _CH, _CH)])
        plsc.subcore_barrier()

        def refill_start(g, ep):
            pltpu.async_copy(rec_h.at[wid, g], e3.at[ep], rsem[ep])
            pltpu.async_copy(w_h.at[wid, g], wv.at[ep], rsem[ep])

        def refill_wait(g, ep):
            pltpu.make_async_copy(rec_h.at[wid, g], e3.at[ep],
                                  rsem[ep]).wait()
            pltpu.make_async_copy(w_h.at[wid, g], wv.at[ep],
                                  rsem[ep]).wait()

        def gather_start(rp, ep):
            pltpu.async_copy(hp_h.at[e3.at[ep, 0]], rows.at[rp], gsem[rp])

        def gather_wait(rp, ep):
            pltpu.make_async_copy(hp_h.at[e3.at[ep, 0]], rows.at[rp],
                                  gsem[rp]).wait()

        def scatter_start(rp, ep):
            pltpu.async_copy(rows.at[rp], acc.at[e3.at[ep, 1]], ssem[rp],
                             add=True)

        def scatter_wait(rp, ep):
            pltpu.make_async_copy(rows.at[rp], acc.at[e3.at[ep, 1]],
                                  ssem[rp]).wait()

        def scale(rp, ep):
            def body(kk, cc):
                w16 = wv[ep, pl.ds(kk * 16, 16)]
                base = kk * 16
                for l in range(16):
                    wvec = jnp.full((16,), w16[l])
                    for j in range(_D // 16):
                        sl = pl.ds(j * 16, 16)
                        rows[rp, base + l, sl] = rows[rp, base + l, sl] * wvec
                return cc

            lax.fori_loop(0, _CH // 16, body, 0)

        for g in range(3):
            refill_start(g, g)
        refill_wait(0, 0)
        gather_start(0, 0)

        def quad(i, carry):
            for kph in range(4):
                ch = 4 * i + kph
                rp = kph % 2
                ep = kph
                epn = (kph + 1) % 4
                epr = (kph + 3) % 4
                gather_wait(rp, ep)

                @pl.when(ch > 0)
                def _():
                    scatter_wait(1 - rp, (kph + 3) % 4)

                @pl.when(ch + 1 < nchunk)
                def _():
                    refill_wait(ch + 1, epn)
                    gather_start(1 - rp, epn)

                scale(rp, ep)
                scatter_start(rp, ep)

                @pl.when(ch + 3 < nchunk)
                def _():
                    refill_start(ch + 3, epr)

            return carry

        lax.fori_loop(0, nchunk // 4, quad, 0)
        if nchunk % 4 == 1:
            gather_wait(0, 0)
            scatter_wait(1, 3)
            scale(0, 0)
            pltpu.sync_copy(rows.at[0], acc.at[e3.at[0, 1]], add=True)
        plsc.subcore_barrier()
        pltpu.sync_copy(acc.at[pl.ds(s * _RPT, _RPT)],
                        out_h.at[c, pl.ds(s * _RPT, _RPT)])

    return k(hp, rec, w2)


def _prep_body(degp, x, wb, h1p, d1o):
    db = degp[0] + degp[1] + 1.0
    d1 = jnp.where(db > 0, 1.0 / jnp.sqrt(db), 0.0)
    d1o[...] = d1
    h1p[...] = d1 * jnp.dot(x[...], wb[...], preferred_element_type=jnp.float32)


def _prep(degp, x, wb):
    r = 400
    return pl.pallas_call(
        _prep_body,
        grid=(_N // r,),
        in_specs=[
            pl.BlockSpec((2, r, 1), lambda i: (0, i, 0)),
            pl.BlockSpec((r, _D), lambda i: (i, 0)),
            pl.BlockSpec((_D, _D), lambda i: (0, 0)),
        ],
        out_specs=[
            pl.BlockSpec((r, _D), lambda i: (i, 0)),
            pl.BlockSpec((r, 1), lambda i: (i, 0)),
        ],
        out_shape=[
            jax.ShapeDtypeStruct((_N, _D), jnp.float32),
            jax.ShapeDtypeStruct((_N, 1), jnp.float32),
        ],
    )(degp, x, wb)


def _mid_body(sp, h1p, d1, degt, x, bb, sg, sb, wt, s2o, h2po, d2o):
    u = d1[...] * (sp[0] + sp[1] + h1p[...]) + bb[...] + x[...]
    u = jnp.maximum(_ln(u, sg[...], sb[...]), 0.0)
    s2o[...] = u
    dt = degt[0] + degt[1] + 1.0
    d2 = jnp.where(dt > 0, 1.0 / jnp.sqrt(dt), 0.0)
    d2o[...] = d2
    h2po[...] = d2 * jnp.dot(u, wt[...], preferred_element_type=jnp.float32)


def _mid(sp, h1p, d1, degt, x, bb, sg, sb, wt):
    r = 400
    return pl.pallas_call(
        _mid_body,
        grid=(_N // r,),
        in_specs=[
            pl.BlockSpec((2, r, _D), lambda i: (0, i, 0)),
            pl.BlockSpec((r, _D), lambda i: (i, 0)),
            pl.BlockSpec((r, 1), lambda i: (i, 0)),
            pl.BlockSpec((2, r, 1), lambda i: (0, i, 0)),
            pl.BlockSpec((r, _D), lambda i: (i, 0)),
            pl.BlockSpec((1, _D), lambda i: (0, 0)),
            pl.BlockSpec((1, _D), lambda i: (0, 0)),
            pl.BlockSpec((1, _D), lambda i: (0, 0)),
            pl.BlockSpec((_D, _D), lambda i: (0, 0)),
        ],
        out_specs=[
            pl.BlockSpec((r, _D), lambda i: (i, 0)),
            pl.BlockSpec((r, _D), lambda i: (i, 0)),
            pl.BlockSpec((r, 1), lambda i: (i, 0)),
        ],
        out_shape=[
            jax.ShapeDtypeStruct((_N, _D), jnp.float32),
            jax.ShapeDtypeStruct((_N, _D), jnp.float32),
            jax.ShapeDtypeStruct((_N, 1), jnp.float32),
        ],
    )(sp, h1p, d1, degt, x, bb.reshape(1, _D), sg.reshape(1, _D),
      sb.reshape(1, _D), wt)


def _final_body(sp, h2p, d2, s2, x, bt, tg, tb, sg, sb, out):
    u = d2[...] * (sp[0] + sp[1] + h2p[...]) + bt[...] + s2[...]
    u = jnp.maximum(_ln(u, tg[...], tb[...]), 0.0)
    out[...] = _ln(u + x[...], sg[...], sb[...])


def _final(sp, h2p, d2, s2, x, bt, tg, tb, sg, sb):
    r = 400
    return pl.pallas_call(
        _final_body,
        grid=(_N // r,),
        in_specs=[
            pl.BlockSpec((2, r, _D), lambda i: (0, i, 0)),
            pl.BlockSpec((r, _D), lambda i: (i, 0)),
            pl.BlockSpec((r, 1), lambda i: (i, 0)),
            pl.BlockSpec((r, _D), lambda i: (i, 0)),
            pl.BlockSpec((r, _D), lambda i: (i, 0)),
            pl.BlockSpec((1, _D), lambda i: (0, 0)),
            pl.BlockSpec((1, _D), lambda i: (0, 0)),
            pl.BlockSpec((1, _D), lambda i: (0, 0)),
            pl.BlockSpec((1, _D), lambda i: (0, 0)),
            pl.BlockSpec((1, _D), lambda i: (0, 0)),
        ],
        out_specs=pl.BlockSpec((r, _D), lambda i: (i, 0)),
        out_shape=jax.ShapeDtypeStruct((_N, _D), jnp.float32),
    )(sp, h2p, d2, s2, x, bt.reshape(1, _D), tg.reshape(1, _D),
      tb.reshape(1, _D), sg.reshape(1, _D), sb.reshape(1, _D))


def kernel(x_struct, x_gene, bold_edge_index, bold_edge_attr,
           temporal_edge_index, temporal_edge_attr, time_steps,
           W_bold, b_bold, W_temp, b_temp, P1, pb1, ln_g, ln_b, P2, pb2,
           struct_g, struct_b, temp_g, temp_b):
    bw = bold_edge_attr[:, 0]

    bs2 = bold_edge_index[0].reshape(_NW, -1, _CH)
    bd2 = bold_edge_index[1].reshape(_NW, -1, _CH)
    ts2 = temporal_edge_index[0].reshape(_NW, -1, _CH)
    td2 = temporal_edge_index[1].reshape(_NW, -1, _CH)
    bw2 = bw.reshape(_NW, -1, _CH)

    brec = jnp.stack([bs2, bd2], axis=2)

    degb = _sc_degree(bd2, bw2)[:, :, None]
    h1p, d1 = _prep(degb, x_struct, W_bold)
    sb_p = _sc_messages(h1p, brec, bw2)

    tw = _edge_mlp(temporal_edge_attr, P1, pb1, ln_g, ln_b, P2, pb2)[:, 0]
    tw2 = tw.reshape(_NW, -1, _CH)
    trec = jnp.stack([ts2, td2], axis=2)
    tw2b, _ = lax.optimization_barrier((tw2, sb_p))
    degt = _sc_degree(td2, tw2b)[:, :, None]

    struct2, h2p, d2 = _mid(sb_p, h1p, d1, degt, x_struct, b_bold,
                            struct_g, struct_b, W_temp)

    st_p = _sc_messages(h2p, trec, tw2)
    out = _final(st_p, h2p, d2, struct2, x_struct, b_temp,
                 temp_g, temp_b, struct_g, struct_b)
    return (out, out)


# speedup vs baseline: 1.1443x; 1.0054x over previous
import functools

import jax
import jax.numpy as jnp
from jax import lax
from jax.experimental import pallas as pl
from jax.experimental.pallas import tpu as pltpu
from jax.experimental.pallas import tpu_sc as plsc

_N = 10000
_D = 128
_EPS = 1e-5

_NC = 2
_NS = 16
_NW = _NC * _NS
_CH = 80
_NP = 10240
_RPT = _NP // _NS


def _ln(x, g, b):
    m = jnp.mean(x, axis=-1, keepdims=True)
    xm = x - m
    v = jnp.mean(xm * xm, axis=-1, keepdims=True)
    return xm * (1.0 / jnp.sqrt(v + _EPS)) * g + b


def _mlp_body(tea, p1, pb1, g, b, p2, pb2, out):
    h = jnp.dot(tea[...], p1[...], preferred_element_type=jnp.float32) + pb1[...]
    h = jnp.maximum(_ln(h, g[...], b[...]), 0.0)
    out[...] = jnp.dot(h, p2[...], preferred_element_type=jnp.float32) + pb2[...]


def _edge_mlp(tea, p1, pb1, g, b, p2, pb2):
    e, k = tea.shape
    be = 16000
    return pl.pallas_call(
        _mlp_body,
        grid=(e // be,),
        in_specs=[
            pl.BlockSpec((be, k), lambda i: (i, 0)),
            pl.BlockSpec((k, _D), lambda i: (0, 0)),
            pl.BlockSpec((1, _D), lambda i: (0, 0)),
            pl.BlockSpec((1, _D), lambda i: (0, 0)),
            pl.BlockSpec((1, _D), lambda i: (0, 0)),
            pl.BlockSpec((_D, 1), lambda i: (0, 0)),
            pl.BlockSpec((1, 1), lambda i: (0, 0)),
        ],
        out_specs=pl.BlockSpec((be, 1), lambda i: (i, 0)),
        out_shape=jax.ShapeDtypeStruct((e, 1), jnp.float32),
    )(tea, p1, pb1.reshape(1, _D), g.reshape(1, _D), b.reshape(1, _D),
      p2, pb2.reshape(1, 1))


def _sc_degree(d2, w2):
    nchunk = d2.shape[1]

    @functools.partial(
        pl.kernel,
        out_type=jax.ShapeDtypeStruct((_NC, _NP), jnp.float32),
        mesh=plsc.VectorSubcoreMesh(core_axis_name="c", subcore_axis_name="s"),
        scratch_types=[
            pltpu.VMEM((nchunk, _CH), jnp.int32),
            pltpu.VMEM((nchunk, _CH), jnp.float32),
            pltpu.VMEM((_RPT,), jnp.float32),
            pltpu.VMEM_SHARED((_NP,), jnp.float32),
        ],
    )
    def k(d_h, w_h, out_h, dix, wv, zb, acc):
        c = lax.axis_index("c")
        s = lax.axis_index("s")
        wid = s * _NC + c
        for i in range(_RPT // 16):
            zb[pl.ds(i * 16, 16)] = jnp.zeros((16,), jnp.float32)
        pltpu.sync_copy(zb, acc.at[pl.ds(s * _RPT, _RPT)])
        pltpu.sync_copy(d_h.at[wid], dix)
        pltpu.sync_copy(w_h.at[wid], wv)
        plsc.subcore_barrier()

        def chunk(g, carry):
            pltpu.sync_copy(wv.at[g], acc.at[dix.at[g]], add=True)
            return carry

        lax.fori_loop(0, nchunk, chunk, 0)
        plsc.subcore_barrier()
        pltpu.sync_copy(acc.at[pl.ds(s * _RPT, _RPT)],
                        out_h.at[c, pl.ds(s * _RPT, _RPT)])

    return k(d2, w2)


def _sc_messages(hp, rec, w2):
    nchunk = rec.shape[1]

    @functools.partial(
        pl.kernel,
        out_type=jax.ShapeDtypeStruct((_NC, _NP, _D), jnp.float32),
        mesh=plsc.VectorSubcoreMesh(core_axis_name="c", subcore_axis_name="s"),
        scratch_types=[
            pltpu.VMEM((4, 2, _CH), jnp.int32),
            pltpu.VMEM((4, _CH), jnp.float32),
            pltpu.VMEM((2, _CH, _D), jnp.float32),
            pltpu.VMEM_SHARED((_NP, _D), jnp.float32),
            [pltpu.SemaphoreType.DMA] * 2,
            [pltpu.SemaphoreType.DMA] * 2,
            [pltpu.SemaphoreType.DMA] * 4,
        ],
    )
    def k(hp_h, rec_h, w_h, out_h, e3, wv, rows, acc, gsem, ssem, rsem):
        c = lax.axis_index("c")
        s = lax.axis_index("s")
        wid = s * _NC + c

        def zrow(e, carry):
            for j in range(_D // 16):
                rows[0, e, pl.ds(j * 16, 16)] = jnp.zeros((16,), jnp.float32)
            return carry

        lax.fori_loop(0, _CH, zrow, 0)
        for kk in range(_RPT // _CH):
            pltpu.sync_copy(rows.at[0],
                            acc.at[pl.ds(s * _RPT + kk * _CH, _CH)])
        plsc.subcore_barrier()

        def refill_start(g, ep):
            pltpu.async_copy(rec_h.at[wid, g], e3.at[ep], rsem[ep])
            pltpu.async_copy(w_h.at[wid, g], wv.at[ep], rsem[ep])

        def refill_wait(g, ep):
            pltpu.make_async_copy(rec_h.at[wid, g], e3.at[ep],
                                  rsem[ep]).wait()
            pltpu.make_async_copy(w_h.at[wid, g], wv.at[ep],
                                  rsem[ep]).wait()

        def gather_start(rp, ep):
            pltpu.async_copy(hp_h.at[e3.at[ep, 0]], rows.at[rp], gsem[rp])

        def gather_wait(rp, ep):
            pltpu.make_async_copy(hp_h.at[e3.at[ep, 0]], rows.at[rp],
                                  gsem[rp]).wait()

        def scatter_start(rp, ep):
            pltpu.async_copy(rows.at[rp], acc.at[e3.at[ep, 1]], ssem[rp],
                             add=True)

        def scatter_wait(rp, ep):
            pltpu.make_async_copy(rows.at[rp], acc.at[e3.at[ep, 1]],
                                  ssem[rp]).wait()

        def scale(rp, ep):
            def body(kk, cc):
                w16 = wv[ep, pl.ds(kk * 16, 16)]
                base = kk * 16
                for l in range(16):
                    wvec = jnp.full((16,), w16[l])
                    for j in range(_D // 16):
                        sl = pl.ds(j * 16, 16)
                        rows[rp, base + l, sl] = rows[rp, base + l, sl] * wvec
                return cc

            lax.fori_loop(0, _CH // 16, body, 0)

        for g in range(3):
            refill_start(g, g)
        refill_wait(0, 0)
        gather_start(0, 0)

        def quad(i, carry):
            for kph in range(4):
                ch = 4 * i + kph
                rp = kph % 2
                ep = kph
                epn = (kph + 1) % 4
                epr = (kph + 3) % 4
                gather_wait(rp, ep)

                @pl.when(ch > 0)
                def _():
                    scatter_wait(1 - rp, (kph + 3) % 4)

                @pl.when(ch + 1 < nchunk)
                def _():
                    refill_wait(ch + 1, epn)
                    gather_start(1 - rp, epn)

                scale(rp, ep)
                scatter_start(rp, ep)

                @pl.when(ch + 3 < nchunk)
                def _():
                    refill_start(ch + 3, epr)

            return carry

        lax.fori_loop(0, nchunk // 4, quad, 0)
        if nchunk % 4 == 1:
            gather_wait(0, 0)
            scatter_wait(1, 3)
            scale(0, 0)
            pltpu.sync_copy(rows.at[0], acc.at[e3.at[0, 1]], add=True)
        plsc.subcore_barrier()
        pltpu.sync_copy(acc.at[pl.ds(s * _RPT, _RPT)],
                        out_h.at[c, pl.ds(s * _RPT, _RPT)])

    return k(hp, rec, w2)


def _prep_body(degp, x, wb, h1p, d1o):
    db = degp[0] + degp[1] + 1.0
    d1 = jnp.where(db > 0, 1.0 / jnp.sqrt(db), 0.0)
    d1o[...] = d1
    h1p[...] = d1 * jnp.dot(x[...], wb[...], preferred_element_type=jnp.float32)


def _prep(degp, x, wb):
    r = 400
    return pl.pallas_call(
        _prep_body,
        grid=(_N // r,),
        in_specs=[
            pl.BlockSpec((2, r, 1), lambda i: (0, i, 0)),
            pl.BlockSpec((r, _D), lambda i: (i, 0)),
            pl.BlockSpec((_D, _D), lambda i: (0, 0)),
        ],
        out_specs=[
            pl.BlockSpec((r, _D), lambda i: (i, 0)),
            pl.BlockSpec((r, 1), lambda i: (i, 0)),
        ],
        out_shape=[
            jax.ShapeDtypeStruct((_N, _D), jnp.float32),
            jax.ShapeDtypeStruct((_N, 1), jnp.float32),
        ],
    )(degp, x, wb)


def _mid_body(sp, h1p, d1, degt, x, bb, sg, sb, wt, s2o, h2po, d2o):
    u = d1[...] * (sp[0] + sp[1] + h1p[...]) + bb[...] + x[...]
    u = jnp.maximum(_ln(u, sg[...], sb[...]), 0.0)
    s2o[...] = u
    dt = degt[0] + degt[1] + 1.0
    d2 = jnp.where(dt > 0, 1.0 / jnp.sqrt(dt), 0.0)
    d2o[...] = d2
    h2po[...] = d2 * jnp.dot(u, wt[...], preferred_element_type=jnp.float32)


def _mid(sp, h1p, d1, degt, x, bb, sg, sb, wt):
    r = 400
    return pl.pallas_call(
        _mid_body,
        grid=(_N // r,),
        in_specs=[
            pl.BlockSpec((2, r, _D), lambda i: (0, i, 0)),
            pl.BlockSpec((r, _D), lambda i: (i, 0)),
            pl.BlockSpec((r, 1), lambda i: (i, 0)),
            pl.BlockSpec((2, r, 1), lambda i: (0, i, 0)),
            pl.BlockSpec((r, _D), lambda i: (i, 0)),
            pl.BlockSpec((1, _D), lambda i: (0, 0)),
            pl.BlockSpec((1, _D), lambda i: (0, 0)),
            pl.BlockSpec((1, _D), lambda i: (0, 0)),
            pl.BlockSpec((_D, _D), lambda i: (0, 0)),
        ],
        out_specs=[
            pl.BlockSpec((r, _D), lambda i: (i, 0)),
            pl.BlockSpec((r, _D), lambda i: (i, 0)),
            pl.BlockSpec((r, 1), lambda i: (i, 0)),
        ],
        out_shape=[
            jax.ShapeDtypeStruct((_N, _D), jnp.float32),
            jax.ShapeDtypeStruct((_N, _D), jnp.float32),
            jax.ShapeDtypeStruct((_N, 1), jnp.float32),
        ],
    )(sp, h1p, d1, degt, x, bb.reshape(1, _D), sg.reshape(1, _D),
      sb.reshape(1, _D), wt)


def _final_body(sp, h2p, d2, s2, x, bt, tg, tb, sg, sb, out):
    u = d2[...] * (sp[0] + sp[1] + h2p[...]) + bt[...] + s2[...]
    u = jnp.maximum(_ln(u, tg[...], tb[...]), 0.0)
    out[...] = _ln(u + x[...], sg[...], sb[...])


def _final(sp, h2p, d2, s2, x, bt, tg, tb, sg, sb):
    r = 400
    return pl.pallas_call(
        _final_body,
        grid=(_N // r,),
        in_specs=[
            pl.BlockSpec((2, r, _D), lambda i: (0, i, 0)),
            pl.BlockSpec((r, _D), lambda i: (i, 0)),
            pl.BlockSpec((r, 1), lambda i: (i, 0)),
            pl.BlockSpec((r, _D), lambda i: (i, 0)),
            pl.BlockSpec((r, _D), lambda i: (i, 0)),
            pl.BlockSpec((1, _D), lambda i: (0, 0)),
            pl.BlockSpec((1, _D), lambda i: (0, 0)),
            pl.BlockSpec((1, _D), lambda i: (0, 0)),
            pl.BlockSpec((1, _D), lambda i: (0, 0)),
            pl.BlockSpec((1, _D), lambda i: (0, 0)),
        ],
        out_specs=pl.BlockSpec((r, _D), lambda i: (i, 0)),
        out_shape=jax.ShapeDtypeStruct((_N, _D), jnp.float32),
    )(sp, h2p, d2, s2, x, bt.reshape(1, _D), tg.reshape(1, _D),
      tb.reshape(1, _D), sg.reshape(1, _D), sb.reshape(1, _D))


def kernel(x_struct, x_gene, bold_edge_index, bold_edge_attr,
           temporal_edge_index, temporal_edge_attr, time_steps,
           W_bold, b_bold, W_temp, b_temp, P1, pb1, ln_g, ln_b, P2, pb2,
           struct_g, struct_b, temp_g, temp_b):
    bw = bold_edge_attr[:, 0]

    bs2 = bold_edge_index[0].reshape(_NW, -1, _CH)
    bd2 = bold_edge_index[1].reshape(_NW, -1, _CH)
    ts2 = temporal_edge_index[0].reshape(_NW, -1, _CH)
    td2 = temporal_edge_index[1].reshape(_NW, -1, _CH)
    bw2 = bw.reshape(_NW, -1, _CH)

    brec = jnp.stack([bs2, bd2], axis=2)

    degb = _sc_degree(bd2, bw2)[:, :, None]
    h1p, d1 = _prep(degb, x_struct, W_bold)
    sb_p = _sc_messages(h1p, brec, bw2)

    tw = _edge_mlp(temporal_edge_attr, P1, pb1, ln_g, ln_b, P2, pb2)[:, 0]
    tw2 = tw.reshape(_NW, -1, _CH)
    trec = jnp.stack([ts2, td2], axis=2)
    tw2b, _ = lax.optimization_barrier((tw2, sb_p))
    degt = _sc_degree(td2, tw2b)[:, :, None]

    struct2, h2p, d2 = _mid(sb_p, h1p, d1, degt, x_struct, b_bold,
                            struct_g, struct_b, W_temp)

    st_p = _sc_messages(h2p, trec, tw2)
    out = _final(st_p, h2p, d2, struct2, x_struct, b_temp,
                 temp_g, temp_b, struct_g, struct_b)
    return (out, out)
